# baseline (speedup 1.0000x reference)
import functools

import jax
import jax.numpy as jnp
from jax import lax
from jax.experimental import pallas as pl
from jax.experimental.pallas import tpu as pltpu
from jax.experimental.pallas import tpu_sc as plsc

NUM_OBS = 1000
NUM_ACT = 100
D = 32
B = 4096

NC = 2
NS = 16
L = 16
NW = NC * NS
B_PER_W = B // NW


_sc_mesh = plsc.VectorSubcoreMesh(
    core_axis_name="c", subcore_axis_name="s", num_cores=NC, num_subcores=NS
)


@functools.partial(
    pl.kernel,
    out_type=(
        jax.ShapeDtypeStruct((B, D), jnp.float32),
        jax.ShapeDtypeStruct((B, D), jnp.float32),
    ),
    mesh=_sc_mesh,
    compiler_params=pltpu.CompilerParams(needs_layout_passes=False),
    scratch_types=[
        pltpu.VMEM((B_PER_W,), jnp.int32),
        pltpu.VMEM((B_PER_W,), jnp.int32),
        pltpu.VMEM((B_PER_W,), jnp.int32),
        pltpu.VMEM((B_PER_W,), jnp.int32),
        pltpu.VMEM((B_PER_W, D), jnp.float32),
        pltpu.VMEM((B_PER_W, D), jnp.float32),
        pltpu.SemaphoreType.DMA,
        pltpu.SemaphoreType.DMA,
    ],
)
def _sc_gather(obs_hbm, act_hbm, fobs_hbm, fact_hbm, wf_hbm, wb_hbm,
               fwd_hbm, bwd_hbm,
               idxf_v, idxb_v, actf_v, actb_v, rows_f, rows_b, semf, semb):
    wid = lax.axis_index("s") * NC + lax.axis_index("c")
    base = wid * B_PER_W
    sl = pl.ds(base, B_PER_W)
    pltpu.sync_copy(obs_hbm.at[sl], idxf_v)
    pltpu.sync_copy(act_hbm.at[sl], actf_v)
    pltpu.sync_copy(fobs_hbm.at[sl], idxb_v)
    pltpu.sync_copy(fact_hbm.at[sl], actb_v)
    for i in range(B_PER_W // L):
        v = pl.ds(i * L, L)
        idxf_v[v] = (jnp.clip(idxf_v[v], 0, NUM_OBS - 1) * NUM_ACT
                     + jnp.clip(actf_v[v], 0, NUM_ACT - 1))
        idxb_v[v] = (jnp.clip(idxb_v[v], 0, NUM_OBS - 1) * NUM_ACT
                     + jnp.clip(actb_v[v], 0, NUM_ACT - 1))
    lane = jax.lax.iota(jnp.int32, L)

    def fire(j, carry):
        c = (j // L) * L
        k = j - c
        chf = idxf_v[pl.ds(c, L)]
        chb = idxb_v[pl.ds(c, L)]
        rf = jnp.sum(jnp.where(lane == k, chf, 0))
        rb = jnp.sum(jnp.where(lane == k, chb, 0))
        pltpu.async_copy(wf_hbm.at[pl.ds(rf, 1)], rows_f.at[pl.ds(j, 1)], semf)
        pltpu.async_copy(wb_hbm.at[pl.ds(rb, 1)], rows_b.at[pl.ds(j, 1)], semb)
        return carry

    lax.fori_loop(0, B_PER_W, fire, 0)
    pltpu.make_async_copy(wf_hbm.at[pl.ds(0, B_PER_W)], rows_f, semf).wait()
    pltpu.make_async_copy(wb_hbm.at[pl.ds(0, B_PER_W)], rows_b, semb).wait()
    pltpu.sync_copy(rows_f, fwd_hbm.at[sl])
    pltpu.sync_copy(rows_b, bwd_hbm.at[sl])


def _mm_body(a_ref, b_ref, o_ref):
    o_ref[...] = lax.dot_general(
        a_ref[...], b_ref[...],
        (((1,), (1,)), ((), ())),
        preferred_element_type=jnp.float32,
    )


_ROWS_PER_STEP = 1024


def _matmul(fwd, bwd):
    return pl.pallas_call(
        _mm_body,
        grid=(B // _ROWS_PER_STEP,),
        in_specs=[
            pl.BlockSpec((_ROWS_PER_STEP, D), lambda i: (i, 0)),
            pl.BlockSpec((B, D), lambda i: (0, 0)),
        ],
        out_specs=pl.BlockSpec((_ROWS_PER_STEP, B), lambda i: (i, 0)),
        out_shape=jax.ShapeDtypeStruct((B, B), jnp.float32),
    )(fwd, bwd)


def kernel(observations, actions, future_observations, future_actions,
           W_forward, W_backward):
    obs = observations.astype(jnp.int32)
    act = actions.astype(jnp.int32)
    fobs = future_observations.astype(jnp.int32)
    fact = future_actions.astype(jnp.int32)
    fwd, bwd = _sc_gather(obs, act, fobs, fact, W_forward, W_backward)
    return _matmul(fwd, bwd)


# speedup vs baseline: 1.0199x; 1.0199x over previous
import functools

import jax
import jax.numpy as jnp
from jax import lax
from jax.experimental import pallas as pl
from jax.experimental.pallas import tpu as pltpu
from jax.experimental.pallas import tpu_sc as plsc

NUM_OBS = 1000
NUM_ACT = 100
D = 32
B = 4096

NC = 2
NS = 16
L = 16
NW = NC * NS
B_PER_W = B // NW


_sc_mesh = plsc.VectorSubcoreMesh(
    core_axis_name="c", subcore_axis_name="s", num_cores=NC, num_subcores=NS
)


@functools.partial(
    pl.kernel,
    out_type=(
        jax.ShapeDtypeStruct((B, D), jnp.float32),
        jax.ShapeDtypeStruct((B, D), jnp.float32),
    ),
    mesh=_sc_mesh,
    compiler_params=pltpu.CompilerParams(needs_layout_passes=False),
    scratch_types=[
        pltpu.VMEM((B_PER_W,), jnp.int32),
        pltpu.VMEM((B_PER_W,), jnp.int32),
        pltpu.VMEM((B_PER_W,), jnp.int32),
        pltpu.VMEM((B_PER_W,), jnp.int32),
        pltpu.VMEM((B_PER_W, D), jnp.float32),
        pltpu.VMEM((B_PER_W, D), jnp.float32),
        pltpu.SemaphoreType.DMA,
        pltpu.SemaphoreType.DMA,
    ],
)
def _sc_gather(obs_hbm, act_hbm, fobs_hbm, fact_hbm, wf_hbm, wb_hbm,
               fwd_hbm, bwd_hbm,
               idxf_v, idxb_v, actf_v, actb_v, rows_f, rows_b, semf, semb):
    wid = lax.axis_index("s") * NC + lax.axis_index("c")
    base = wid * B_PER_W
    sl = pl.ds(base, B_PER_W)
    pltpu.sync_copy(obs_hbm.at[sl], idxf_v)
    pltpu.sync_copy(act_hbm.at[sl], actf_v)
    pltpu.sync_copy(fobs_hbm.at[sl], idxb_v)
    pltpu.sync_copy(fact_hbm.at[sl], actb_v)
    for i in range(B_PER_W // L):
        v = pl.ds(i * L, L)
        idxf_v[v] = (jnp.clip(idxf_v[v], 0, NUM_OBS - 1) * NUM_ACT
                     + jnp.clip(actf_v[v], 0, NUM_ACT - 1))
        idxb_v[v] = (jnp.clip(idxb_v[v], 0, NUM_OBS - 1) * NUM_ACT
                     + jnp.clip(actb_v[v], 0, NUM_ACT - 1))
    lane = jax.lax.iota(jnp.int32, L)

    def fire(j, carry):
        c = (j // L) * L
        k = j - c
        chf = idxf_v[pl.ds(c, L)]
        chb = idxb_v[pl.ds(c, L)]
        rf = jnp.sum(jnp.where(lane == k, chf, 0))
        rb = jnp.sum(jnp.where(lane == k, chb, 0))
        pltpu.async_copy(wf_hbm.at[pl.ds(rf, 1)], rows_f.at[pl.ds(j, 1)], semf)
        pltpu.async_copy(wb_hbm.at[pl.ds(rb, 1)], rows_b.at[pl.ds(j, 1)], semb)
        return carry

    lax.fori_loop(0, B_PER_W, fire, 0)
    pltpu.make_async_copy(wf_hbm.at[pl.ds(0, B_PER_W)], rows_f, semf).wait()
    pltpu.make_async_copy(wb_hbm.at[pl.ds(0, B_PER_W)], rows_b, semb).wait()
    pltpu.sync_copy(rows_f, fwd_hbm.at[sl])
    pltpu.sync_copy(rows_b, bwd_hbm.at[sl])


def _mm_body(a_ref, b_ref, o_ref):
    o_ref[...] = lax.dot_general(
        a_ref[...], b_ref[...],
        (((1,), (1,)), ((), ())),
        preferred_element_type=jnp.float32,
    )


_ROWS_PER_STEP = 512


def _matmul(fwd, bwd):
    return pl.pallas_call(
        _mm_body,
        grid=(B // _ROWS_PER_STEP,),
        in_specs=[
            pl.BlockSpec((_ROWS_PER_STEP, D), lambda i: (i, 0)),
            pl.BlockSpec((B, D), lambda i: (0, 0)),
        ],
        out_specs=pl.BlockSpec((_ROWS_PER_STEP, B), lambda i: (i, 0)),
        out_shape=jax.ShapeDtypeStruct((B, B), jnp.float32),
    )(fwd, bwd)


def kernel(observations, actions, future_observations, future_actions,
           W_forward, W_backward):
    obs = observations.astype(jnp.int32)
    act = actions.astype(jnp.int32)
    fobs = future_observations.astype(jnp.int32)
    fact = future_actions.astype(jnp.int32)
    fwd, bwd = _sc_gather(obs, act, fobs, fact, W_forward, W_backward)
    return _matmul(fwd, bwd)
